# Initial kernel scaffold; baseline (speedup 1.0000x reference)
import jax
import jax.numpy as jnp
from jax.experimental import pallas as pl


def kernel(x, edge_index, Wl1, bl1, Wr1, Wl2, bl2, Wr2):
    raise NotImplementedError("write your pallas kernel here")


# speedup vs baseline: 5.9414x; 5.9414x over previous
import jax
import jax.numpy as jnp
from jax import lax
from jax.experimental import pallas as pl
from jax.experimental.pallas import tpu as pltpu
from jax.experimental.pallas import tpu_sc as plsc

N = 10000
E = 320000
D = 128
NC = 2
NS = 16
NW = NC * NS
EPW = E // NW
C = 80
NCHUNK = EPW // C
NPAD = 10240
RPT = NPAD // NS
L = 16

ROWS_BLK = 1000
GRID_R = N // ROWS_BLK


def _mm_body(x_ref, w_ref, o_ref):
    o_ref[...] = lax.dot_general(x_ref[...], w_ref[...],
                                 (((1,), (1,)), ((), ())),
                                 preferred_element_type=jnp.float32)


def _mm(x, w):
    return pl.pallas_call(
        _mm_body,
        grid=(GRID_R,),
        in_specs=[
            pl.BlockSpec((ROWS_BLK, D), lambda i: (i, 0)),
            pl.BlockSpec((D, D), lambda i: (0, 0)),
        ],
        out_specs=pl.BlockSpec((ROWS_BLK, D), lambda i: (i, 0)),
        out_shape=jax.ShapeDtypeStruct((N, D), jnp.float32),
    )(x, w)


def _inv_cnt_body(c_ref, o_ref):
    s = jnp.sum(c_ref[...], axis=0)
    o_ref[...] = (1.0 / jnp.maximum(s, 1.0))[:, None]


def _inv_cnt(cnts):
    return pl.pallas_call(
        _inv_cnt_body,
        out_shape=jax.ShapeDtypeStruct((NPAD, 1), jnp.float32),
    )(cnts)


def _combine1_body(part_ref, inv_ref, x_ref, wr_ref, b_ref, wl2_ref,
                   h_ref, p2_ref):
    agg = part_ref[0] + part_ref[1]
    mean_lin = agg * inv_ref[...]
    xr = lax.dot_general(x_ref[...], wr_ref[...], (((1,), (1,)), ((), ())),
                         preferred_element_type=jnp.float32)
    h = jnp.maximum(mean_lin + b_ref[...] + xr, 0.0)
    h_ref[...] = h
    p2_ref[...] = lax.dot_general(h, wl2_ref[...], (((1,), (1,)), ((), ())),
                                  preferred_element_type=jnp.float32)


def _combine1(parts, inv, x, wr, b, wl2):
    return pl.pallas_call(
        _combine1_body,
        grid=(GRID_R,),
        in_specs=[
            pl.BlockSpec((NC, ROWS_BLK, D), lambda i: (0, i, 0)),
            pl.BlockSpec((ROWS_BLK, 1), lambda i: (i, 0)),
            pl.BlockSpec((ROWS_BLK, D), lambda i: (i, 0)),
            pl.BlockSpec((D, D), lambda i: (0, 0)),
            pl.BlockSpec((1, D), lambda i: (0, 0)),
            pl.BlockSpec((D, D), lambda i: (0, 0)),
        ],
        out_specs=[
            pl.BlockSpec((ROWS_BLK, D), lambda i: (i, 0)),
            pl.BlockSpec((ROWS_BLK, D), lambda i: (i, 0)),
        ],
        out_shape=[
            jax.ShapeDtypeStruct((N, D), jnp.float32),
            jax.ShapeDtypeStruct((N, D), jnp.float32),
        ],
    )(parts, inv, x, wr, b, wl2)


def _combine2_body(part_ref, inv_ref, h_ref, wr_ref, b_ref, o_ref):
    agg = part_ref[0] + part_ref[1]
    mean_lin = agg * inv_ref[...]
    hr = lax.dot_general(h_ref[...], wr_ref[...], (((1,), (1,)), ((), ())),
                         preferred_element_type=jnp.float32)
    o_ref[...] = mean_lin + b_ref[...] + hr


def _combine2(parts, inv, h, wr, b):
    return pl.pallas_call(
        _combine2_body,
        grid=(GRID_R,),
        in_specs=[
            pl.BlockSpec((NC, ROWS_BLK, D), lambda i: (0, i, 0)),
            pl.BlockSpec((ROWS_BLK, 1), lambda i: (i, 0)),
            pl.BlockSpec((ROWS_BLK, D), lambda i: (i, 0)),
            pl.BlockSpec((D, D), lambda i: (0, 0)),
            pl.BlockSpec((1, D), lambda i: (0, 0)),
        ],
        out_specs=pl.BlockSpec((ROWS_BLK, D), lambda i: (i, 0)),
        out_shape=jax.ShapeDtypeStruct((N, D), jnp.float32),
    )(parts, inv, h, wr, b)


def _sc_agg_body(p_hbm, src_hbm, dst_hbm, zero_hbm, out_hbm, cnt_hbm,
                 src_v, dst_v, rows_v, cnt_v, acc_sh):
    cid = lax.axis_index("c")
    sid = lax.axis_index("s")
    wid = cid * NS + sid

    pltpu.sync_copy(zero_hbm, acc_sh.at[pl.ds(sid * RPT, RPT)])

    zeros16 = jnp.zeros((L,), jnp.float32)

    def zinit(i, carry):
        cnt_v[pl.ds(i * L, L)] = zeros16
        return carry

    lax.fori_loop(0, NPAD // L, zinit, 0)
    plsc.subcore_barrier()

    ebase = wid * EPW
    ones16 = jnp.ones((L,), jnp.float32)

    def chunk(j, carry):
        base = ebase + j * C
        pltpu.sync_copy(src_hbm.at[pl.ds(base, C)], src_v)
        pltpu.sync_copy(dst_hbm.at[pl.ds(base, C)], dst_v)
        pltpu.sync_copy(p_hbm.at[src_v], rows_v)
        pltpu.sync_copy(rows_v, acc_sh.at[dst_v], add=True)
        for k in range(C // L):
            idx = dst_v[pl.ds(k * L, L)]
            plsc.addupdate_scatter(cnt_v, [idx], ones16)
        return carry

    lax.fori_loop(0, NCHUNK, chunk, 0)
    plsc.subcore_barrier()

    pltpu.sync_copy(acc_sh.at[pl.ds(sid * RPT, RPT)],
                    out_hbm.at[cid, pl.ds(sid * RPT, RPT)])
    pltpu.sync_copy(cnt_v, cnt_hbm.at[wid])


def _sc_agg(p, src, dst, zeros):
    mesh = plsc.VectorSubcoreMesh(core_axis_name="c", subcore_axis_name="s")
    kern = pl.kernel(
        _sc_agg_body,
        out_type=(
            jax.ShapeDtypeStruct((NC, NPAD, D), jnp.float32),
            jax.ShapeDtypeStruct((NW, NPAD), jnp.float32),
        ),
        mesh=mesh,
        scratch_types=[
            pltpu.VMEM((C,), jnp.int32),
            pltpu.VMEM((C,), jnp.int32),
            pltpu.VMEM((C, D), jnp.float32),
            pltpu.VMEM((NPAD,), jnp.float32),
            pltpu.VMEM_SHARED((NPAD, D), jnp.float32),
        ],
        compiler_params=pltpu.CompilerParams(needs_layout_passes=False),
    )
    return kern(p, src, dst, zeros)


def kernel(x, edge_index, Wl1, bl1, Wr1, Wl2, bl2, Wr2):
    ei = edge_index.astype(jnp.int32)
    src = ei[0]
    dst = ei[1]
    zeros = jnp.zeros((RPT, D), jnp.float32)
    bl1r = bl1.reshape(1, D)
    bl2r = bl2.reshape(1, D)

    p1 = _mm(x, Wl1)
    parts1, cnts = _sc_agg(p1, src, dst, zeros)
    inv = _inv_cnt(cnts)
    h, p2 = _combine1(parts1, inv, x, Wr1, bl1r, Wl2)
    parts2, _ = _sc_agg(p2, src, dst, zeros)
    out = _combine2(parts2, inv, h, Wr2, bl2r)
    return out
